# baseline (speedup 1.0000x reference)
import functools

import jax
import jax.numpy as jnp
from jax import lax
from jax.experimental import pallas as pl
from jax.experimental.pallas import tpu as pltpu
from jax.experimental.pallas import tpu_sc as plsc

_VOCAB = 1000
_K = 8
_EMBED = 32
_B = 4096
_HID = 512
_OUT = 1000

_NC = 2
_NS = 16
_NW = _NC * _NS
_BG = _B * _K
_B_PER_W = _BG // _NW
_CHUNK = 128
_NCHUNK = _B_PER_W // _CHUNK

_BB_ARG = 256
_BB_MLP = 512


def _argmax_body(x_ref, idx_ref):
    xb = x_ref[...]
    cols = []
    for k in range(_K):
        xk = xb[:, k * _VOCAB:(k + 1) * _VOCAB]
        m = jnp.max(xk, axis=1, keepdims=True)
        ii = lax.broadcasted_iota(jnp.int32, xk.shape, 1)
        cand = jnp.where(xk == m, ii, _VOCAB)
        cols.append(jnp.min(cand, axis=1, keepdims=True))
    idx_ref[...] = jnp.concatenate(cols, axis=1)


def _argmax_indices(x):
    grid = _B // _BB_ARG
    return pl.pallas_call(
        _argmax_body,
        grid=(grid,),
        in_specs=[pl.BlockSpec((_BB_ARG, _K * _VOCAB), lambda i: (i, 0))],
        out_specs=pl.BlockSpec((_BB_ARG, _K), lambda i: (i, 0)),
        out_shape=jax.ShapeDtypeStruct((_B, _K), jnp.int32),
    )(x)


@functools.lru_cache(maxsize=1)
def _make_sc_gather():
    mesh = plsc.VectorSubcoreMesh(core_axis_name="c", subcore_axis_name="s")

    @functools.partial(
        pl.kernel,
        mesh=mesh,
        out_type=jax.ShapeDtypeStruct((_NW, _B_PER_W, _EMBED), jnp.float32),
        scratch_types=[
            pltpu.VMEM((_NCHUNK, _CHUNK), jnp.int32),
            pltpu.VMEM((_B_PER_W, _EMBED), jnp.float32),
            pltpu.SemaphoreType.DMA,
        ],
        compiler_params=pltpu.CompilerParams(use_tc_tiling_on_sc=False),
    )
    def _sc_gather(emb_hbm, idx_hbm, out_hbm, idx_v, rows_v, sem):
        wid = lax.axis_index("s") * _NC + lax.axis_index("c")
        pltpu.sync_copy(idx_hbm.at[wid], idx_v)
        copies = []
        for j in range(_NCHUNK):
            copies.append(
                pltpu.async_copy(
                    emb_hbm.at[idx_v.at[j]],
                    rows_v.at[pl.ds(j * _CHUNK, _CHUNK)],
                    sem,
                )
            )
        for cp in copies:
            cp.wait()
        pltpu.sync_copy(rows_v, out_hbm.at[wid])

    return _sc_gather


def _mlp_body(fe_ref, w1_ref, b1_ref, w2_ref, b2_ref, o_ref):
    h = jnp.dot(fe_ref[...], w1_ref[...], preferred_element_type=jnp.float32)
    h = jnp.maximum(h + b1_ref[...], 0.0)
    o = jnp.dot(h, w2_ref[...], preferred_element_type=jnp.float32)
    o_ref[...] = o + b2_ref[...]


def _mlp(fe, w1, b1, w2, b2):
    grid = _B // _BB_MLP
    return pl.pallas_call(
        _mlp_body,
        grid=(grid,),
        in_specs=[
            pl.BlockSpec((_BB_MLP, _K * _EMBED), lambda i: (i, 0)),
            pl.BlockSpec((_K * _EMBED, _HID), lambda i: (0, 0)),
            pl.BlockSpec((1, _HID), lambda i: (0, 0)),
            pl.BlockSpec((_HID, _OUT), lambda i: (0, 0)),
            pl.BlockSpec((1, _OUT), lambda i: (0, 0)),
        ],
        out_specs=pl.BlockSpec((_BB_MLP, _OUT), lambda i: (i, 0)),
        out_shape=jax.ShapeDtypeStruct((_B, _OUT), jnp.float32),
    )(fe, w1, b1, w2, b2)


def kernel(x, emb, W1, b1, W2, b2):
    idx = _argmax_indices(x)
    idx_w = idx.reshape(_NW, _NCHUNK, _CHUNK)
    rows = _make_sc_gather()(emb, idx_w)
    fe = rows.reshape(_B, _K * _EMBED)
    return _mlp(fe, W1, b1.reshape(1, _HID), W2, b2.reshape(1, _OUT))


# speedup vs baseline: 1.0271x; 1.0271x over previous
import functools

import jax
import jax.numpy as jnp
from jax import lax
from jax.experimental import pallas as pl
from jax.experimental.pallas import tpu as pltpu
from jax.experimental.pallas import tpu_sc as plsc

_VOCAB = 1000
_K = 8
_EMBED = 32
_B = 4096
_HID = 512
_OUT = 1000

_NC = 2
_NS = 16
_NW = _NC * _NS
_BG = _B * _K
_B_PER_W = _BG // _NW
_CHUNK = 128
_NCHUNK = _B_PER_W // _CHUNK

_BB_ARG = 512
_BB_MLP = 512


def _argmax_body(x_ref, idx_ref):
    xb = x_ref[...]
    cols = []
    for k in range(_K):
        xk = xb[:, k * _VOCAB:(k + 1) * _VOCAB]
        m = jnp.max(xk, axis=1, keepdims=True)
        ii = lax.broadcasted_iota(jnp.int32, xk.shape, 1)
        cand = jnp.where(xk == m, ii, _VOCAB)
        cols.append(jnp.min(cand, axis=1, keepdims=True))
    idx_ref[...] = jnp.concatenate(cols, axis=1)


def _argmax_indices(x):
    grid = _B // _BB_ARG
    return pl.pallas_call(
        _argmax_body,
        grid=(grid,),
        in_specs=[pl.BlockSpec((_BB_ARG, _K * _VOCAB), lambda i: (i, 0))],
        out_specs=pl.BlockSpec((_BB_ARG, _K), lambda i: (i, 0)),
        out_shape=jax.ShapeDtypeStruct((_B, _K), jnp.int32),
    )(x)


@functools.lru_cache(maxsize=1)
def _make_sc_gather():
    mesh = plsc.VectorSubcoreMesh(core_axis_name="c", subcore_axis_name="s")

    @functools.partial(
        pl.kernel,
        mesh=mesh,
        out_type=jax.ShapeDtypeStruct((_NW, _B_PER_W, _EMBED), jnp.float32),
        scratch_types=[
            pltpu.VMEM((_NCHUNK, _CHUNK), jnp.int32),
            pltpu.VMEM((_B_PER_W, _EMBED), jnp.float32),
            pltpu.SemaphoreType.DMA,
        ],
        compiler_params=pltpu.CompilerParams(use_tc_tiling_on_sc=False),
    )
    def _sc_gather(emb_hbm, idx_hbm, out_hbm, idx_v, rows_v, sem):
        wid = lax.axis_index("s") * _NC + lax.axis_index("c")
        pltpu.sync_copy(idx_hbm.at[wid], idx_v)
        copies = []
        for j in range(_NCHUNK):
            copies.append(
                pltpu.async_copy(
                    emb_hbm.at[idx_v.at[j]],
                    rows_v.at[pl.ds(j * _CHUNK, _CHUNK)],
                    sem,
                )
            )
        for cp in copies:
            cp.wait()
        pltpu.sync_copy(rows_v, out_hbm.at[wid])

    return _sc_gather


def _mlp_body(fe_ref, w1_ref, b1_ref, w2_ref, b2_ref, o_ref):
    h = jnp.dot(fe_ref[...], w1_ref[...], preferred_element_type=jnp.float32)
    h = jnp.maximum(h + b1_ref[...], 0.0)
    o = jnp.dot(h, w2_ref[...], preferred_element_type=jnp.float32)
    o_ref[...] = o + b2_ref[...]


def _mlp(fe, w1, b1, w2, b2):
    grid = _B // _BB_MLP
    return pl.pallas_call(
        _mlp_body,
        grid=(grid,),
        in_specs=[
            pl.BlockSpec((_BB_MLP, _K * _EMBED), lambda i: (i, 0)),
            pl.BlockSpec((_K * _EMBED, _HID), lambda i: (0, 0)),
            pl.BlockSpec((1, _HID), lambda i: (0, 0)),
            pl.BlockSpec((_HID, _OUT), lambda i: (0, 0)),
            pl.BlockSpec((1, _OUT), lambda i: (0, 0)),
        ],
        out_specs=pl.BlockSpec((_BB_MLP, _OUT), lambda i: (i, 0)),
        out_shape=jax.ShapeDtypeStruct((_B, _OUT), jnp.float32),
    )(fe, w1, b1, w2, b2)


def kernel(x, emb, W1, b1, W2, b2):
    idx = _argmax_indices(x)
    idx_w = idx.reshape(_NW, _NCHUNK, _CHUNK)
    rows = _make_sc_gather()(emb, idx_w)
    fe = rows.reshape(_B, _K * _EMBED)
    return _mlp(fe, W1, b1.reshape(1, _HID), W2, b2.reshape(1, _OUT))


# speedup vs baseline: 1.1969x; 1.1653x over previous
import functools

import jax
import jax.numpy as jnp
from jax import lax
from jax.experimental import pallas as pl
from jax.experimental.pallas import tpu as pltpu
from jax.experimental.pallas import tpu_sc as plsc

_VOCAB = 1000
_K = 8
_EMBED = 32
_B = 4096
_HID = 512
_OUT = 1000

_NC = 2
_NS = 16
_NW = _NC * _NS
_BG = _B * _K
_B_PER_W = _BG // _NW
_CHUNK = 128
_NCHUNK = _B_PER_W // _CHUNK

_BB_ARG = 512
_BB_MLP = 512


def _argmax_body(x_ref, idx_ref):
    xb = x_ref[...]
    cols = []
    for k in range(_K):
        xk = xb[:, k * _VOCAB:(k + 1) * _VOCAB]
        m = jnp.max(xk, axis=1, keepdims=True)
        ii = lax.broadcasted_iota(jnp.int32, xk.shape, 1)
        cand = jnp.where(xk == m, ii, _VOCAB)
        cols.append(jnp.min(cand, axis=1, keepdims=True))
    idx_ref[...] = jnp.concatenate(cols, axis=1)


def _argmax_indices(x):
    grid = _B // _BB_ARG
    return pl.pallas_call(
        _argmax_body,
        grid=(grid,),
        in_specs=[pl.BlockSpec((_BB_ARG, _K * _VOCAB), lambda i: (i, 0))],
        out_specs=pl.BlockSpec((_BB_ARG, _K), lambda i: (i, 0)),
        out_shape=jax.ShapeDtypeStruct((_B, _K), jnp.int32),
    )(x)


@functools.lru_cache(maxsize=1)
def _make_sc_gather():
    mesh = plsc.VectorSubcoreMesh(core_axis_name="c", subcore_axis_name="s")

    @functools.partial(
        pl.kernel,
        mesh=mesh,
        out_type=jax.ShapeDtypeStruct((_NW, _B_PER_W, _EMBED), jnp.float32),
        scratch_types=[
            pltpu.VMEM((_NCHUNK, _CHUNK), jnp.int32),
            pltpu.VMEM((_B_PER_W, _EMBED), jnp.float32),
            pltpu.SemaphoreType.DMA,
        ],
        compiler_params=pltpu.CompilerParams(use_tc_tiling_on_sc=False),
    )
    def _sc_gather(emb_hbm, idx_hbm, out_hbm, idx_v, rows_v, sem):
        wid = lax.axis_index("s") * _NC + lax.axis_index("c")
        pltpu.sync_copy(idx_hbm.at[wid], idx_v)
        copies = []
        for j in range(_NCHUNK):
            copies.append(
                pltpu.async_copy(
                    emb_hbm.at[idx_v.at[j]],
                    rows_v.at[pl.ds(j * _CHUNK, _CHUNK)],
                    sem,
                )
            )
        for cp in copies:
            cp.wait()
        pltpu.sync_copy(rows_v, out_hbm.at[wid])

    return _sc_gather


def _fused_body(x_ref, emb_ref, w1_ref, b1_ref, w2_ref, b2_ref, o_ref):
    xb = x_ref[...]
    emb = emb_ref[...]
    fe_parts = []
    for k in range(_K):
        xk = xb[:, k * _VOCAB:(k + 1) * _VOCAB]
        m = jnp.max(xk, axis=1, keepdims=True)
        ii = lax.broadcasted_iota(jnp.int32, xk.shape, 1)
        cand = jnp.where(xk == m, ii, _VOCAB)
        idxk = jnp.min(cand, axis=1, keepdims=True)
        onehot = (ii == idxk).astype(jnp.float32)
        fe_parts.append(
            jnp.dot(onehot, emb, preferred_element_type=jnp.float32))
    fe = jnp.concatenate(fe_parts, axis=1)
    h = jnp.dot(fe, w1_ref[...], preferred_element_type=jnp.float32)
    h = jnp.maximum(h + b1_ref[...], 0.0)
    o = jnp.dot(h, w2_ref[...], preferred_element_type=jnp.float32)
    o_ref[...] = o + b2_ref[...]


def _fused(x, emb, w1, b1, w2, b2, bb):
    grid = x.shape[0] // bb
    return pl.pallas_call(
        _fused_body,
        grid=(grid,),
        in_specs=[
            pl.BlockSpec((bb, _K * _VOCAB), lambda i: (i, 0)),
            pl.BlockSpec((_VOCAB, _EMBED), lambda i: (0, 0)),
            pl.BlockSpec((_K * _EMBED, _HID), lambda i: (0, 0)),
            pl.BlockSpec((1, _HID), lambda i: (0, 0)),
            pl.BlockSpec((_HID, _OUT), lambda i: (0, 0)),
            pl.BlockSpec((1, _OUT), lambda i: (0, 0)),
        ],
        out_specs=pl.BlockSpec((bb, _OUT), lambda i: (i, 0)),
        out_shape=jax.ShapeDtypeStruct((x.shape[0], _OUT), jnp.float32),
    )(x, emb, w1, b1, w2, b2)


def _mlp_body(fe_ref, w1_ref, b1_ref, w2_ref, b2_ref, o_ref):
    h = jnp.dot(fe_ref[...], w1_ref[...], preferred_element_type=jnp.float32)
    h = jnp.maximum(h + b1_ref[...], 0.0)
    o = jnp.dot(h, w2_ref[...], preferred_element_type=jnp.float32)
    o_ref[...] = o + b2_ref[...]


def _mlp(fe, w1, b1, w2, b2):
    grid = _B // _BB_MLP
    return pl.pallas_call(
        _mlp_body,
        grid=(grid,),
        in_specs=[
            pl.BlockSpec((_BB_MLP, _K * _EMBED), lambda i: (i, 0)),
            pl.BlockSpec((_K * _EMBED, _HID), lambda i: (0, 0)),
            pl.BlockSpec((1, _HID), lambda i: (0, 0)),
            pl.BlockSpec((_HID, _OUT), lambda i: (0, 0)),
            pl.BlockSpec((1, _OUT), lambda i: (0, 0)),
        ],
        out_specs=pl.BlockSpec((_BB_MLP, _OUT), lambda i: (i, 0)),
        out_shape=jax.ShapeDtypeStruct((_B, _OUT), jnp.float32),
    )(fe, w1, b1, w2, b2)


def kernel(x, emb, W1, b1, W2, b2):
    return _fused(x, emb, W1, b1.reshape(1, _HID), W2, b2.reshape(1, _OUT), 512)
